# baseline (speedup 1.0000x reference)
import functools

import jax
import jax.numpy as jnp
from jax import lax
from jax.experimental import pallas as pl
from jax.experimental.pallas import tpu as pltpu
from jax.experimental.pallas import tpu_sc as plsc

N = 100000
D = 128
TWO = 2 * D
B = 512
C = 10

R = 5000
K = N // R


def _main_kernel(hi_ref, he_ref, b_ref, gw1_ref, gb1_ref, gw2_ref, gb2_ref,
                 pw_ref, pb_ref, pv_ref, cw1_ref, cb1_ref, cw2_ref, cb2_ref,
                 hf_out, e_out, l_out, emb_out, logits_out):
    i = pl.program_id(0)
    k = pl.num_programs(0) - 1

    @pl.when(i == 0)
    def _init():
        l_out[...] = jnp.zeros_like(l_out)
        emb_out[...] = jnp.zeros_like(emb_out)

    @pl.when(i < k)
    def _main():
        z = jnp.concatenate([hi_ref[...], he_ref[...]], axis=1)
        zb = z.astype(jnp.bfloat16)
        h1 = jax.lax.dot_general(zb, gw1_ref[...], (((1,), (1,)), ((), ())),
                                 preferred_element_type=jnp.float32) + gb1_ref[...]
        h = 0.5 * h1 * (1.0 + jax.lax.erf(h1 * 0.7071067811865476))
        g = jax.nn.sigmoid(
            jax.lax.dot_general(h.astype(jnp.bfloat16), gw2_ref[...],
                                (((1,), (1,)), ((), ())),
                                preferred_element_type=jnp.float32)
            + gb2_ref[...])
        hf = g * z
        hf_out[...] = hf
        hfb = hf.astype(jnp.bfloat16)
        t = jnp.tanh(
            jax.lax.dot_general(hfb, pw_ref[...], (((1,), (1,)), ((), ())),
                                preferred_element_type=jnp.float32)
            + pb_ref[...])
        s_col = jax.lax.dot_general(t.astype(jnp.bfloat16), pv_ref[...],
                                    (((1,), (0,)), ((), ())),
                                    preferred_element_type=jnp.float32)
        e_col = jnp.exp(s_col)
        e_out[0, 0, :] = e_col[:, 0]

        b = b_ref[0, 0, :]
        seg = jax.lax.broadcasted_iota(jnp.int32, (R, B), 1)
        mask = b[:, None] == seg

        l_part = jnp.sum(jnp.where(mask, e_col, 0.0), axis=0)
        l_out[0, :] = l_out[0, :] + l_part
        p = mask.astype(jnp.bfloat16)
        contrib = jax.lax.dot_general(p, (hf * e_col).astype(jnp.bfloat16),
                                      (((0,), (0,)), ((), ())),
                                      preferred_element_type=jnp.float32)
        emb_out[...] = emb_out[...] + contrib

    @pl.when(i == k)
    def _cls():
        ge = emb_out[...] / (l_out[0, :][:, None] + 1e-12)
        emb_out[...] = ge
        h2 = jax.nn.relu(
            jax.lax.dot_general(ge, cw1_ref[...], (((1,), (1,)), ((), ())),
                                preferred_element_type=jnp.float32)
            + cb1_ref[...])
        logits_out[...] = jax.lax.dot_general(
            h2, cw2_ref[...], (((1,), (1,)), ((), ())),
            preferred_element_type=jnp.float32) + cb2_ref[...]


_NC = 2
_NS = 16
_NW = _NC * _NS
_N_SC = 102400
_RW = _N_SC // _NW


def _attn_sc_kernel(e_hbm, b_hbm, l_hbm, out_hbm, l_v, e_v, b_v, o_v):
    wid = lax.axis_index("s") * _NC + lax.axis_index("c")
    base = wid * _RW
    pltpu.sync_copy(l_hbm, l_v)
    pltpu.sync_copy(e_hbm.at[pl.ds(base, _RW)], e_v)
    pltpu.sync_copy(b_hbm.at[pl.ds(base, _RW)], b_v)

    def body(j, carry):
        idx = b_v[pl.ds(j * 16, 16)]
        ev = e_v[pl.ds(j * 16, 16)]
        lv = plsc.load_gather(l_v, [idx])
        o_v[pl.ds(j * 16, 16)] = ev / (lv + 1e-12)
        return carry

    lax.fori_loop(0, _RW // 16, body, 0)
    pltpu.sync_copy(o_v, out_hbm.at[pl.ds(base, _RW)])


@functools.partial(jax.jit, donate_argnums=())
def kernel(H_intra, H_inter, batch, gate_W1, gate_b1, gate_W2, gate_b2,
           poolW_W, poolW_b, pool_w, cls_W1, cls_b1, cls_W2, cls_b2):
    b32 = batch.astype(jnp.int32)
    b3d = b32.reshape(K, 1, R)

    row_spec = pl.BlockSpec((R, D), lambda i: (jnp.minimum(i, K - 1), 0))
    vec_spec = pl.BlockSpec((1, 1, R), lambda i: (jnp.minimum(i, K - 1), 0, 0))
    full = lambda shp: pl.BlockSpec(shp, lambda i: tuple(0 for _ in shp))

    hf, e, l, emb, logits = pl.pallas_call(
        _main_kernel,
        grid=(K + 1,),
        in_specs=[
            row_spec, row_spec, vec_spec,
            full((TWO, TWO)), full((1, TWO)),
            full((TWO, TWO)), full((1, TWO)),
            full((TWO, TWO)), full((1, TWO)),
            full((TWO, 1)),
            full((TWO, TWO)), full((1, TWO)),
            full((C, TWO)), full((1, C)),
        ],
        out_specs=[
            pl.BlockSpec((R, TWO), lambda i: (jnp.minimum(i, K - 1), 0)),
            vec_spec,
            full((1, B)),
            full((B, TWO)), full((B, C)),
        ],
        out_shape=[
            jax.ShapeDtypeStruct((N, TWO), jnp.float32),
            jax.ShapeDtypeStruct((K, 1, R), jnp.float32),
            jax.ShapeDtypeStruct((1, B), jnp.float32),
            jax.ShapeDtypeStruct((B, TWO), jnp.float32),
            jax.ShapeDtypeStruct((B, C), jnp.float32),
        ],
    )(H_intra, H_inter, b3d,
      gate_W1.astype(jnp.bfloat16), gate_b1.reshape(1, TWO),
      gate_W2.astype(jnp.bfloat16), gate_b2.reshape(1, TWO),
      poolW_W.astype(jnp.bfloat16), poolW_b.reshape(1, TWO),
      pool_w.reshape(TWO, 1).astype(jnp.bfloat16),
      cls_W1, cls_b1.reshape(1, TWO),
      cls_W2, cls_b2.reshape(1, C))

    e_pad = jnp.pad(e.reshape(N), (0, _N_SC - N))
    b_pad = jnp.pad(b32, (0, _N_SC - N), constant_values=B)
    l_pad = jnp.pad(l.reshape(B), (0, B))
    attn = pl.kernel(
        _attn_sc_kernel,
        out_type=jax.ShapeDtypeStruct((_N_SC,), jnp.float32),
        mesh=plsc.VectorSubcoreMesh(core_axis_name="c", subcore_axis_name="s"),
        compiler_params=pltpu.CompilerParams(needs_layout_passes=False),
        scratch_types=[
            pltpu.VMEM((2 * B,), jnp.float32),
            pltpu.VMEM((_RW,), jnp.float32),
            pltpu.VMEM((_RW,), jnp.int32),
            pltpu.VMEM((_RW,), jnp.float32),
        ],
    )(e_pad, b_pad, l_pad)

    return (logits, emb, attn[:N], hf)


# speedup vs baseline: 1.0690x; 1.0690x over previous
import functools

import jax
import jax.numpy as jnp
from jax import lax
from jax.experimental import pallas as pl
from jax.experimental.pallas import tpu as pltpu
from jax.experimental.pallas import tpu_sc as plsc

N = 100000
D = 128
TWO = 2 * D
B = 512
C = 10

R = 5000
K = N // R


W = 128


def _main_kernel(bounds_ref, hi_ref, he_ref, b_ref, gw1_ref, gb1_ref,
                 gw2_ref, gb2_ref, pw_ref, pb_ref, pv_ref, cw1_ref, cb1_ref,
                 cw2_ref, cb2_ref, hf_out, e_out, l_out, emb_out, logits_out):
    i = pl.program_id(0)
    k = pl.num_programs(0) - 1

    @pl.when(i == 0)
    def _init():
        l_out[...] = jnp.zeros_like(l_out)
        emb_out[...] = jnp.zeros_like(emb_out)

    @pl.when(i < k)
    def _main():
        z = jnp.concatenate([hi_ref[...], he_ref[...]], axis=1)
        zb = z.astype(jnp.bfloat16)
        h1 = jax.lax.dot_general(zb, gw1_ref[...], (((1,), (1,)), ((), ())),
                                 preferred_element_type=jnp.float32) + gb1_ref[...]
        h = 0.5 * h1 * (1.0 + jax.lax.erf(h1 * 0.7071067811865476))
        g = jax.nn.sigmoid(
            jax.lax.dot_general(h.astype(jnp.bfloat16), gw2_ref[...],
                                (((1,), (1,)), ((), ())),
                                preferred_element_type=jnp.float32)
            + gb2_ref[...])
        hf = g * z
        hf_out[...] = hf
        hfb = hf.astype(jnp.bfloat16)
        t = jnp.tanh(
            jax.lax.dot_general(hfb, pw_ref[...], (((1,), (1,)), ((), ())),
                                preferred_element_type=jnp.float32)
            + pb_ref[...])
        s_col = jax.lax.dot_general(t.astype(jnp.bfloat16), pv_ref[...],
                                    (((1,), (0,)), ((), ())),
                                    preferred_element_type=jnp.float32)
        e_col = jnp.exp(s_col)
        e_out[0, 0, :] = e_col[:, 0]

        b = b_ref[0, 0, :]
        eb = e_col.astype(jnp.bfloat16)
        whf = (hf * e_col).astype(jnp.bfloat16)

        b0 = bounds_ref[0, 0, 0]
        b1 = bounds_ref[0, 0, 1]
        start = jnp.minimum((b0 // 8) * 8, B - W)
        narrow = (b1 - start) < W

        @pl.when(narrow)
        def _narrow():
            rel = b - start
            segw = jax.lax.broadcasted_iota(jnp.int32, (R, W), 1)
            p = (rel[:, None] == segw).astype(jnp.bfloat16)
            contrib = jax.lax.dot_general(p, whf, (((0,), (0,)), ((), ())),
                                          preferred_element_type=jnp.float32)
            l_c = jax.lax.dot_general(p, eb, (((0,), (0,)), ((), ())),
                                      preferred_element_type=jnp.float32)
            emb_out[pl.ds(start, W), :] = emb_out[pl.ds(start, W), :] + contrib
            l_out[pl.ds(start, W), :] = l_out[pl.ds(start, W), :] + l_c

        @pl.when(jnp.logical_not(narrow))
        def _wide():
            seg = jax.lax.broadcasted_iota(jnp.int32, (R, B), 1)
            p = (b[:, None] == seg).astype(jnp.bfloat16)
            contrib = jax.lax.dot_general(p, whf, (((0,), (0,)), ((), ())),
                                          preferred_element_type=jnp.float32)
            l_c = jax.lax.dot_general(p, eb, (((0,), (0,)), ((), ())),
                                      preferred_element_type=jnp.float32)
            emb_out[...] = emb_out[...] + contrib
            l_out[...] = l_out[...] + l_c

    @pl.when(i == k)
    def _cls():
        ge = emb_out[...] / (l_out[...] + 1e-12)
        emb_out[...] = ge
        h2 = jax.nn.relu(
            jax.lax.dot_general(ge, cw1_ref[...], (((1,), (1,)), ((), ())),
                                preferred_element_type=jnp.float32)
            + cb1_ref[...])
        logits_out[...] = jax.lax.dot_general(
            h2, cw2_ref[...], (((1,), (1,)), ((), ())),
            preferred_element_type=jnp.float32) + cb2_ref[...]


_NC = 2
_NS = 16
_NW = _NC * _NS
_N_SC = 102400
_RW = _N_SC // _NW


def _attn_sc_kernel(e_hbm, b_hbm, l_hbm, out_hbm, l_v, e_v, b_v, o_v):
    wid = lax.axis_index("s") * _NC + lax.axis_index("c")
    base = wid * _RW
    pltpu.sync_copy(l_hbm, l_v)
    pltpu.sync_copy(e_hbm.at[pl.ds(base, _RW)], e_v)
    pltpu.sync_copy(b_hbm.at[pl.ds(base, _RW)], b_v)

    def body(j, carry):
        idx = b_v[pl.ds(j * 16, 16)]
        ev = e_v[pl.ds(j * 16, 16)]
        lv = plsc.load_gather(l_v, [idx])
        o_v[pl.ds(j * 16, 16)] = ev / (lv + 1e-12)
        return carry

    lax.fori_loop(0, _RW // 16, body, 0)
    pltpu.sync_copy(o_v, out_hbm.at[pl.ds(base, _RW)])


@functools.partial(jax.jit, donate_argnums=())
def kernel(H_intra, H_inter, batch, gate_W1, gate_b1, gate_W2, gate_b2,
           poolW_W, poolW_b, pool_w, cls_W1, cls_b1, cls_W2, cls_b2):
    b32 = batch.astype(jnp.int32)
    b3d = b32.reshape(K, 1, R)
    bounds = jnp.stack([b32[::R], b32[R - 1::R]], axis=1).reshape(K, 1, 2)

    row_spec = pl.BlockSpec((R, D), lambda i: (jnp.minimum(i, K - 1), 0))
    vec_spec = pl.BlockSpec((1, 1, R), lambda i: (jnp.minimum(i, K - 1), 0, 0))
    full = lambda shp: pl.BlockSpec(shp, lambda i: tuple(0 for _ in shp))

    hf, e, l, emb, logits = pl.pallas_call(
        _main_kernel,
        grid=(K + 1,),
        in_specs=[
            pl.BlockSpec((1, 1, 2), lambda i: (jnp.minimum(i, K - 1), 0, 0),
                         memory_space=pltpu.SMEM),
            row_spec, row_spec, vec_spec,
            full((TWO, TWO)), full((1, TWO)),
            full((TWO, TWO)), full((1, TWO)),
            full((TWO, TWO)), full((1, TWO)),
            full((TWO, 1)),
            full((TWO, TWO)), full((1, TWO)),
            full((C, TWO)), full((1, C)),
        ],
        out_specs=[
            pl.BlockSpec((R, TWO), lambda i: (jnp.minimum(i, K - 1), 0)),
            vec_spec,
            full((B, 1)),
            full((B, TWO)), full((B, C)),
        ],
        out_shape=[
            jax.ShapeDtypeStruct((N, TWO), jnp.float32),
            jax.ShapeDtypeStruct((K, 1, R), jnp.float32),
            jax.ShapeDtypeStruct((B, 1), jnp.float32),
            jax.ShapeDtypeStruct((B, TWO), jnp.float32),
            jax.ShapeDtypeStruct((B, C), jnp.float32),
        ],
    )(bounds, H_intra, H_inter, b3d,
      gate_W1.astype(jnp.bfloat16), gate_b1.reshape(1, TWO),
      gate_W2.astype(jnp.bfloat16), gate_b2.reshape(1, TWO),
      poolW_W.astype(jnp.bfloat16), poolW_b.reshape(1, TWO),
      pool_w.reshape(TWO, 1).astype(jnp.bfloat16),
      cls_W1, cls_b1.reshape(1, TWO),
      cls_W2, cls_b2.reshape(1, C))

    e_pad = jnp.pad(e.reshape(N), (0, _N_SC - N))
    b_pad = jnp.pad(b32, (0, _N_SC - N), constant_values=B)
    l_pad = jnp.pad(l.reshape(B), (0, B))
    attn = pl.kernel(
        _attn_sc_kernel,
        out_type=jax.ShapeDtypeStruct((_N_SC,), jnp.float32),
        mesh=plsc.VectorSubcoreMesh(core_axis_name="c", subcore_axis_name="s"),
        compiler_params=pltpu.CompilerParams(needs_layout_passes=False),
        scratch_types=[
            pltpu.VMEM((2 * B,), jnp.float32),
            pltpu.VMEM((_RW,), jnp.float32),
            pltpu.VMEM((_RW,), jnp.int32),
            pltpu.VMEM((_RW,), jnp.float32),
        ],
    )(e_pad, b_pad, l_pad)

    return (logits, emb, attn[:N], hf)


# speedup vs baseline: 1.1145x; 1.0426x over previous
import functools

import jax
import jax.numpy as jnp
from jax import lax
from jax.experimental import pallas as pl
from jax.experimental.pallas import tpu as pltpu
from jax.experimental.pallas import tpu_sc as plsc

N = 100000
D = 128
TWO = 2 * D
B = 512
C = 10

R = 10000
K = N // R


W = 128


def _main_kernel(bounds_ref, hi_ref, he_ref, b_ref, gw1_ref, gb1_ref,
                 gw2_ref, gb2_ref, pw_ref, pb_ref, pv_ref, cw1_ref, cb1_ref,
                 cw2_ref, cb2_ref, hf_out, e_out, l_out, emb_out, logits_out):
    i = pl.program_id(0)
    k = pl.num_programs(0) - 1

    @pl.when(i == 0)
    def _init():
        l_out[...] = jnp.zeros_like(l_out)
        emb_out[...] = jnp.zeros_like(emb_out)

    @pl.when(i < k)
    def _main():
        z = jnp.concatenate([hi_ref[...], he_ref[...]], axis=1)
        zb = z.astype(jnp.bfloat16)
        h1 = jax.lax.dot_general(zb, gw1_ref[...], (((1,), (1,)), ((), ())),
                                 preferred_element_type=jnp.float32) + gb1_ref[...]
        h = 0.5 * h1 * (1.0 + jax.lax.erf(h1 * 0.7071067811865476))
        g = jax.nn.sigmoid(
            jax.lax.dot_general(h.astype(jnp.bfloat16), gw2_ref[...],
                                (((1,), (1,)), ((), ())),
                                preferred_element_type=jnp.float32)
            + gb2_ref[...])
        hf = g * z
        hf_out[...] = hf
        hfb = hf.astype(jnp.bfloat16)
        t = jnp.tanh(
            jax.lax.dot_general(hfb, pw_ref[...], (((1,), (1,)), ((), ())),
                                preferred_element_type=jnp.float32)
            + pb_ref[...])
        s_col = jax.lax.dot_general(t.astype(jnp.bfloat16), pv_ref[...],
                                    (((1,), (0,)), ((), ())),
                                    preferred_element_type=jnp.float32)
        e_col = jnp.exp(s_col)
        e_out[0, 0, :] = e_col[:, 0]

        b = b_ref[0, 0, :]
        eb = e_col.astype(jnp.bfloat16)
        whf = (hf * e_col).astype(jnp.bfloat16)

        b0 = bounds_ref[0, 0, 0]
        b1 = bounds_ref[0, 0, 1]
        start = jnp.minimum((b0 // 8) * 8, B - W)
        narrow = (b1 - start) < W

        @pl.when(narrow)
        def _narrow():
            rel = b - start
            segw = jax.lax.broadcasted_iota(jnp.int32, (R, W), 1)
            p = (rel[:, None] == segw).astype(jnp.bfloat16)
            contrib = jax.lax.dot_general(p, whf, (((0,), (0,)), ((), ())),
                                          preferred_element_type=jnp.float32)
            l_c = jax.lax.dot_general(p, eb, (((0,), (0,)), ((), ())),
                                      preferred_element_type=jnp.float32)
            emb_out[pl.ds(start, W), :] = emb_out[pl.ds(start, W), :] + contrib
            l_out[pl.ds(start, W), :] = l_out[pl.ds(start, W), :] + l_c

        @pl.when(jnp.logical_not(narrow))
        def _wide():
            seg = jax.lax.broadcasted_iota(jnp.int32, (R, B), 1)
            p = (b[:, None] == seg).astype(jnp.bfloat16)
            contrib = jax.lax.dot_general(p, whf, (((0,), (0,)), ((), ())),
                                          preferred_element_type=jnp.float32)
            l_c = jax.lax.dot_general(p, eb, (((0,), (0,)), ((), ())),
                                      preferred_element_type=jnp.float32)
            emb_out[...] = emb_out[...] + contrib
            l_out[...] = l_out[...] + l_c

    @pl.when(i == k)
    def _cls():
        ge = emb_out[...] / (l_out[...] + 1e-12)
        emb_out[...] = ge
        h2 = jax.nn.relu(
            jax.lax.dot_general(ge, cw1_ref[...], (((1,), (1,)), ((), ())),
                                preferred_element_type=jnp.float32)
            + cb1_ref[...])
        logits_out[...] = jax.lax.dot_general(
            h2, cw2_ref[...], (((1,), (1,)), ((), ())),
            preferred_element_type=jnp.float32) + cb2_ref[...]


_NC = 2
_NS = 16
_NW = _NC * _NS
_N_SC = 102400
_RW = _N_SC // _NW


def _attn_sc_kernel(e_hbm, b_hbm, l_hbm, out_hbm, l_v, e_v, b_v, o_v):
    wid = lax.axis_index("s") * _NC + lax.axis_index("c")
    base = wid * _RW
    pltpu.sync_copy(l_hbm, l_v)
    pltpu.sync_copy(e_hbm.at[pl.ds(base, _RW)], e_v)
    pltpu.sync_copy(b_hbm.at[pl.ds(base, _RW)], b_v)

    def body(j, carry):
        idx = b_v[pl.ds(j * 16, 16)]
        ev = e_v[pl.ds(j * 16, 16)]
        lv = plsc.load_gather(l_v, [idx])
        o_v[pl.ds(j * 16, 16)] = ev / (lv + 1e-12)
        return carry

    lax.fori_loop(0, _RW // 16, body, 0)
    pltpu.sync_copy(o_v, out_hbm.at[pl.ds(base, _RW)])


@functools.partial(jax.jit, donate_argnums=())
def kernel(H_intra, H_inter, batch, gate_W1, gate_b1, gate_W2, gate_b2,
           poolW_W, poolW_b, pool_w, cls_W1, cls_b1, cls_W2, cls_b2):
    b32 = batch.astype(jnp.int32)
    b3d = b32.reshape(K, 1, R)
    bounds = jnp.stack([b32[::R], b32[R - 1::R]], axis=1).reshape(K, 1, 2)

    row_spec = pl.BlockSpec((R, D), lambda i: (jnp.minimum(i, K - 1), 0))
    vec_spec = pl.BlockSpec((1, 1, R), lambda i: (jnp.minimum(i, K - 1), 0, 0))
    full = lambda shp: pl.BlockSpec(shp, lambda i: tuple(0 for _ in shp))

    hf, e, l, emb, logits = pl.pallas_call(
        _main_kernel,
        grid=(K + 1,),
        in_specs=[
            pl.BlockSpec((1, 1, 2), lambda i: (jnp.minimum(i, K - 1), 0, 0),
                         memory_space=pltpu.SMEM),
            row_spec, row_spec, vec_spec,
            full((TWO, TWO)), full((1, TWO)),
            full((TWO, TWO)), full((1, TWO)),
            full((TWO, TWO)), full((1, TWO)),
            full((TWO, 1)),
            full((TWO, TWO)), full((1, TWO)),
            full((C, TWO)), full((1, C)),
        ],
        out_specs=[
            pl.BlockSpec((R, TWO), lambda i: (jnp.minimum(i, K - 1), 0)),
            vec_spec,
            full((B, 1)),
            full((B, TWO)), full((B, C)),
        ],
        out_shape=[
            jax.ShapeDtypeStruct((N, TWO), jnp.float32),
            jax.ShapeDtypeStruct((K, 1, R), jnp.float32),
            jax.ShapeDtypeStruct((B, 1), jnp.float32),
            jax.ShapeDtypeStruct((B, TWO), jnp.float32),
            jax.ShapeDtypeStruct((B, C), jnp.float32),
        ],
    )(bounds, H_intra, H_inter, b3d,
      gate_W1.astype(jnp.bfloat16), gate_b1.reshape(1, TWO),
      gate_W2.astype(jnp.bfloat16), gate_b2.reshape(1, TWO),
      poolW_W.astype(jnp.bfloat16), poolW_b.reshape(1, TWO),
      pool_w.reshape(TWO, 1).astype(jnp.bfloat16),
      cls_W1, cls_b1.reshape(1, TWO),
      cls_W2, cls_b2.reshape(1, C))

    e_pad = jnp.pad(e.reshape(N), (0, _N_SC - N))
    b_pad = jnp.pad(b32, (0, _N_SC - N), constant_values=B)
    l_pad = jnp.pad(l.reshape(B), (0, B))
    attn = pl.kernel(
        _attn_sc_kernel,
        out_type=jax.ShapeDtypeStruct((_N_SC,), jnp.float32),
        mesh=plsc.VectorSubcoreMesh(core_axis_name="c", subcore_axis_name="s"),
        compiler_params=pltpu.CompilerParams(needs_layout_passes=False),
        scratch_types=[
            pltpu.VMEM((2 * B,), jnp.float32),
            pltpu.VMEM((_RW,), jnp.float32),
            pltpu.VMEM((_RW,), jnp.int32),
            pltpu.VMEM((_RW,), jnp.float32),
        ],
    )(e_pad, b_pad, l_pad)

    return (logits, emb, attn[:N], hf)
